# baseline (speedup 1.0000x reference)
import functools

import jax
import jax.numpy as jnp
from jax import lax
from jax.experimental import pallas as pl
from jax.experimental.pallas import tpu as pltpu
from jax.experimental.pallas import tpu_sc as plsc

_L = 16
_N = 32768
_NCHUNK = _N // _L
_R = 128
_NW = 32
_RPW = _R // _NW
_T = 26


def _splat(v):
    return jnp.full((_L,), v, dtype=jnp.float32)


def _bfly_max(v):
    idx = lax.iota(jnp.int32, _L)
    for sh in (8, 4, 2, 1):
        v = jnp.maximum(v, v[idx ^ sh])
    return v


def _bfly_sum(v):
    idx = lax.iota(jnp.int32, _L)
    for sh in (8, 4, 2, 1):
        v = v + v[idx ^ sh]
    return v


def _entmax_row(xbuf, cbuf, clist):
    def max_body(i, acc):
        return jnp.maximum(acc, xbuf[pl.ds(i * _L, _L)])

    macc = lax.fori_loop(0, _NCHUNK, max_body, _splat(-jnp.inf))
    m = _bfly_max(macc)[0] * 0.5
    thr = 2.0 * m - 2.0
    mv = jnp.full((_L,), m, dtype=jnp.float32)

    def filt_body(i, nl):
        v = xbuf[pl.ds(i * _L, _L)]
        cm = _bfly_max(v)[0]
        clist[pl.ds(nl, _L)] = jnp.full((_L,), i, dtype=jnp.int32)
        return nl + (cm > thr).astype(jnp.int32)

    nlist = lax.fori_loop(0, _NCHUNK, filt_body, 0)

    def pack_body(j, c):
        ci = clist[pl.ds(j, _L)][0]
        v = xbuf[pl.ds(ci * _L, _L)]
        cbuf[pl.ds(j * _L, _L)] = v * 0.5 - mv
        return c

    lax.fori_loop(0, nlist, pack_body, 0)

    def newton_iter(_, carry):
        lo, hi, tau = carry

        def acc_body(i, ac):
            accf, accs = ac
            y = jnp.maximum(cbuf[pl.ds(i * _L, _L)] - tau, 0.0)
            return accf + y * y, accs + y

        accf, accs = lax.fori_loop(
            0, nlist, acc_body, (_splat(0.0), _splat(0.0))
        )
        fv = _bfly_sum(accf)
        sv = _bfly_sum(accs)
        gt = fv > 1.0
        lo = jnp.where(gt, tau, lo)
        hi = jnp.where(gt, hi, tau)
        tn = tau + (fv - 1.0) / (2.0 * sv)
        mid = 0.5 * (lo + hi)
        tn = jnp.where((tn > lo) & (tn < hi), tn, mid)
        return lo, hi, tn

    _, _, tau = lax.fori_loop(
        0, _T, newton_iter, (_splat(-1.0), _splat(0.0), _splat(-0.5))
    )

    shift = mv + tau

    def out_body(i, c):
        y = jnp.maximum(xbuf[pl.ds(i * _L, _L)] * 0.5 - shift, 0.0)
        xbuf[pl.ds(i * _L, _L)] = y * y
        return c

    lax.fori_loop(0, _NCHUNK, out_body, 0)


def _make_sc_kernel():
    mesh = plsc.VectorSubcoreMesh(core_axis_name="c", subcore_axis_name="s")

    @functools.partial(
        pl.kernel,
        mesh=mesh,
        out_type=jax.ShapeDtypeStruct((_R, _N), jnp.float32),
        scratch_types=[
            pltpu.VMEM((_N,), jnp.float32),
            pltpu.VMEM((_N,), jnp.float32),
            pltpu.VMEM((_NCHUNK + _L,), jnp.int32),
        ],
    )
    def entmax_sc(scores, out, xbuf, cbuf, clist):
        wid = lax.axis_index("s") * 2 + lax.axis_index("c")

        def row_body(r, c):
            row = wid * _RPW + r
            pltpu.sync_copy(scores.at[row], xbuf)
            _entmax_row(xbuf, cbuf, clist)
            pltpu.sync_copy(xbuf, out.at[row])
            return c

        lax.fori_loop(0, _RPW, row_body, 0)

    return entmax_sc


_entmax_sc = _make_sc_kernel()


def kernel(scores):
    return _entmax_sc(scores)


# speedup vs baseline: 1.9348x; 1.9348x over previous
import functools

import jax
import jax.numpy as jnp
from jax import lax
from jax.experimental import pallas as pl
from jax.experimental.pallas import tpu as pltpu
from jax.experimental.pallas import tpu_sc as plsc

_L = 16
_N = 32768
_NCHUNK = _N // _L
_R = 128
_NW = 32
_RPW = _R // _NW
_T = 40
_EPS = 3e-7
_U1 = 8
_U2 = 4
_U3 = 2


def _splat(v):
    return jnp.full((_L,), v, dtype=jnp.float32)


def _bfly_max(v):
    idx = lax.iota(jnp.int32, _L)
    for sh in (8, 4, 2, 1):
        v = jnp.maximum(v, v[idx ^ sh])
    return v


def _bfly_sum(v):
    idx = lax.iota(jnp.int32, _L)
    for sh in (8, 4, 2, 1):
        v = v + v[idx ^ sh]
    return v


def _entmax_row(xbuf, obuf, clist):
    def max_body(i, acc):
        for k in range(_U1):
            acc = jnp.maximum(acc, xbuf[pl.ds((i * _U1 + k) * _L, _L)])
        return acc

    macc = lax.fori_loop(0, _NCHUNK // _U1, max_body, _splat(-jnp.inf))
    m = _bfly_max(macc)[0] * 0.5
    thr = 2.0 * m - 2.0
    mv = jnp.full((_L,), m, dtype=jnp.float32)

    def filt_body(i, nl):
        for k in range(_U2):
            c = i * _U2 + k
            v = xbuf[pl.ds(c * _L, _L)]
            cm = _bfly_max(v)[0]
            clist[pl.ds(nl, _L)] = jnp.full((_L,), c, dtype=jnp.int32)
            nl = nl + jnp.where(cm > thr, 1, 0)
        return nl

    nlist = lax.fori_loop(0, _NCHUNK // _U2, filt_body, 0)
    clist[pl.ds(nlist, _L)] = jnp.full((_L,), _NCHUNK, dtype=jnp.int32)

    ntrip = (nlist + _U3 - 1) // _U3

    def newton_body(_, carry):
        lo, hi, tau = carry
        live = (hi - lo)[0] > _EPS
        ntrip_eff = jnp.where(live, ntrip, 0)
        shift = mv + tau

        def acc_body(i, ac):
            accf, accs = ac
            for k in range(_U3):
                ci = clist[pl.ds(i * _U3 + k, _L)][0]
                y = jnp.maximum(xbuf[pl.ds(ci * _L, _L)] * 0.5 - shift, 0.0)
                accf = accf + y * y
                accs = accs + y
            return accf, accs

        accf, accs = lax.fori_loop(
            0, ntrip_eff, acc_body, (_splat(0.0), _splat(0.0))
        )
        fv = _bfly_sum(accf)
        sv = _bfly_sum(accs)
        gt = fv > 1.0
        lo2 = jnp.where(gt, tau, lo)
        hi2 = jnp.where(gt, hi, tau)
        tn = tau + (fv - 1.0) / (2.0 * sv)
        mid = 0.5 * (lo2 + hi2)
        tn = jnp.where((tn > lo2) & (tn < hi2), tn, mid)
        g = jnp.full((_L,), jnp.where(live, 1.0, 0.0), dtype=jnp.float32)
        return (
            lo + g * (lo2 - lo),
            hi + g * (hi2 - hi),
            tau + g * (tn - tau),
        )

    _, _, tau = lax.fori_loop(
        0, _T, newton_body, (_splat(-1.0), _splat(0.0), _splat(-0.5))
    )

    shift = mv + tau

    def out_body(i, c):
        ci = clist[pl.ds(i, _L)][0]
        y = jnp.maximum(xbuf[pl.ds(ci * _L, _L)] * 0.5 - shift, 0.0)
        obuf[pl.ds(ci * _L, _L)] = y * y
        return c

    lax.fori_loop(0, nlist, out_body, 0)
    return nlist


def _make_sc_kernel():
    mesh = plsc.VectorSubcoreMesh(core_axis_name="c", subcore_axis_name="s")

    @functools.partial(
        pl.kernel,
        mesh=mesh,
        out_type=jax.ShapeDtypeStruct((_R, _N), jnp.float32),
        scratch_types=[
            pltpu.VMEM((_N + _L,), jnp.float32),
            pltpu.VMEM((_N,), jnp.float32),
            pltpu.VMEM((_NCHUNK + _L,), jnp.int32),
        ],
    )
    def entmax_sc(scores, out, xbuf, obuf, clist):
        wid = lax.axis_index("s") * 2 + lax.axis_index("c")
        xbuf[pl.ds(_N, _L)] = _splat(-1e9)

        def zero_body(i, c):
            for k in range(_U1):
                obuf[pl.ds((i * _U1 + k) * _L, _L)] = _splat(0.0)
            return c

        lax.fori_loop(0, _NCHUNK // _U1, zero_body, 0)

        def row_body(r, c):
            row = wid * _RPW + r
            pltpu.sync_copy(scores.at[row], xbuf.at[pl.ds(0, _N)])
            nlist = _entmax_row(xbuf, obuf, clist)
            pltpu.sync_copy(obuf, out.at[row])

            def rezero_body(i, c2):
                ci = clist[pl.ds(i, _L)][0]
                obuf[pl.ds(ci * _L, _L)] = _splat(0.0)
                return c2

            lax.fori_loop(0, nlist, rezero_body, 0)
            return c

        lax.fori_loop(0, _RPW, row_body, 0)

    return entmax_sc


_entmax_sc = _make_sc_kernel()


def kernel(scores):
    return _entmax_sc(scores)


# speedup vs baseline: 2.2411x; 1.1583x over previous
import functools

import jax
import jax.numpy as jnp
from jax import lax
from jax.experimental import pallas as pl
from jax.experimental.pallas import tpu as pltpu
from jax.experimental.pallas import tpu_sc as plsc

_L = 16
_N = 32768
_NCHUNK = _N // _L
_R = 128
_NW = 32
_RPW = _R // _NW
_T = 40
_EPS = 3e-7
_U1 = 8
_U3 = 4


def _splat(v):
    return jnp.full((_L,), v, dtype=jnp.float32)


def _bfly_max(v):
    idx = lax.iota(jnp.int32, _L)
    for sh in (8, 4, 2, 1):
        v = jnp.maximum(v, v[idx ^ sh])
    return v


def _bfly_sum(v):
    idx = lax.iota(jnp.int32, _L)
    for sh in (8, 4, 2, 1):
        v = v + v[idx ^ sh]
    return v


def _entmax_row(xbuf, obuf, cbuf, clist):
    lanes = lax.iota(jnp.int32, _L)
    pow2 = jnp.left_shift(jnp.ones((_L,), jnp.int32), lanes)

    def max_body(i, acc):
        for k in range(_U1):
            acc = jnp.maximum(acc, xbuf[pl.ds((i * _U1 + k) * _L, _L)])
        return acc

    macc = lax.fori_loop(0, _NCHUNK // _U1, max_body, _splat(-jnp.inf))
    m = _bfly_max(macc)[0] * 0.5
    thr = 2.0 * m - 2.0
    thrv = jnp.full((_L,), thr, dtype=jnp.float32)
    mv = jnp.full((_L,), m, dtype=jnp.float32)

    def filt_body(g, nl):
        cmv = _splat(-jnp.inf)
        for k in range(_L):
            v = xbuf[pl.ds((g * _L + k) * _L, _L)]
            cm = _bfly_max(v)
            cmv = jnp.where(lanes == k, cm, cmv)
        dirty = jnp.where(cmv > thrv, pow2, 0)
        bits = _bfly_sum(dirty)[0]
        base = g * _L
        for k in range(_L):
            clist[pl.ds(nl, _L)] = jnp.full((_L,), base + k, dtype=jnp.int32)
            nl = nl + jnp.bitwise_and(jnp.right_shift(bits, k), 1)
        return nl

    nlist = lax.fori_loop(0, _NCHUNK // _L, filt_body, 0)

    def pack_body(j, c):
        ci = clist[pl.ds(j, _L)][0]
        v = xbuf[pl.ds(ci * _L, _L)]
        cbuf[pl.ds(j * _L, _L)] = v * 0.5 - mv
        return c

    lax.fori_loop(0, nlist, pack_body, 0)
    for k in range(_U3):
        cbuf[pl.ds((nlist + k) * _L, _L)] = _splat(-1e9)
    ntrip = (nlist + _U3 - 1) // _U3

    def newton_body(_, carry):
        lo, hi, tau = carry
        live = (hi - lo)[0] > _EPS
        ntrip_eff = jnp.where(live, ntrip, 0)

        def acc_body(i, ac):
            accf, accs = ac
            for k in range(_U3):
                y = jnp.maximum(cbuf[pl.ds((i * _U3 + k) * _L, _L)] - tau, 0.0)
                accf = accf + y * y
                accs = accs + y
            return accf, accs

        accf, accs = lax.fori_loop(
            0, ntrip_eff, acc_body, (_splat(0.0), _splat(0.0))
        )
        fv = _bfly_sum(accf)
        sv = _bfly_sum(accs)
        gt = fv > 1.0
        lo2 = jnp.where(gt, tau, lo)
        hi2 = jnp.where(gt, hi, tau)
        tn = tau + (fv - 1.0) / (2.0 * sv)
        mid = 0.5 * (lo2 + hi2)
        tn = jnp.where((tn > lo2) & (tn < hi2), tn, mid)
        g = jnp.full((_L,), jnp.where(live, 1.0, 0.0), dtype=jnp.float32)
        return (
            lo + g * (lo2 - lo),
            hi + g * (hi2 - hi),
            tau + g * (tn - tau),
        )

    _, _, tau = lax.fori_loop(
        0, _T, newton_body, (_splat(-1.0), _splat(0.0), _splat(-0.5))
    )

    def out_body(j, c):
        ci = clist[pl.ds(j, _L)][0]
        y = jnp.maximum(cbuf[pl.ds(j * _L, _L)] - tau, 0.0)
        obuf[pl.ds(ci * _L, _L)] = y * y
        return c

    lax.fori_loop(0, nlist, out_body, 0)
    return nlist


def _make_sc_kernel():
    mesh = plsc.VectorSubcoreMesh(core_axis_name="c", subcore_axis_name="s")

    @functools.partial(
        pl.kernel,
        mesh=mesh,
        out_type=jax.ShapeDtypeStruct((_R, _N), jnp.float32),
        scratch_types=[
            pltpu.VMEM((_N,), jnp.float32),
            pltpu.VMEM((_N,), jnp.float32),
            pltpu.VMEM((_N + _L * _U3,), jnp.float32),
            pltpu.VMEM((_NCHUNK + _L,), jnp.int32),
        ],
    )
    def entmax_sc(scores, out, xbuf, obuf, cbuf, clist):
        wid = lax.axis_index("s") * 2 + lax.axis_index("c")

        def zero_body(i, c):
            for k in range(_U1):
                obuf[pl.ds((i * _U1 + k) * _L, _L)] = _splat(0.0)
            return c

        lax.fori_loop(0, _NCHUNK // _U1, zero_body, 0)

        def row_body(r, c):
            row = wid * _RPW + r
            pltpu.sync_copy(scores.at[row], xbuf)
            nlist = _entmax_row(xbuf, obuf, cbuf, clist)
            pltpu.sync_copy(obuf, out.at[row])

            def rezero_body(j, c2):
                ci = clist[pl.ds(j, _L)][0]
                obuf[pl.ds(ci * _L, _L)] = _splat(0.0)
                return c2

            lax.fori_loop(0, nlist, rezero_body, 0)
            return c

        lax.fori_loop(0, _RPW, row_body, 0)

    return entmax_sc


_entmax_sc = _make_sc_kernel()


def kernel(scores):
    return _entmax_sc(scores)


# speedup vs baseline: 2.8087x; 1.2533x over previous
import functools

import jax
import jax.numpy as jnp
from jax import lax
from jax.experimental import pallas as pl
from jax.experimental.pallas import tpu as pltpu
from jax.experimental.pallas import tpu_sc as plsc

_L = 16
_N = 32768
_NCHUNK = _N // _L
_NPAIR = _NCHUNK // 2
_R = 128
_NW = 32
_RPW = _R // _NW
_T = 40
_EPS = 6e-7
_U1 = 8
_U3 = 2
_GP = 8


def _splat(v):
    return jnp.full((_L,), v, dtype=jnp.float32)


def _bfly_max(v):
    idx = lax.iota(jnp.int32, _L)
    for sh in (8, 4, 2, 1):
        v = jnp.maximum(v, v[idx ^ sh])
    return v


def _bfly_sum(v):
    idx = lax.iota(jnp.int32, _L)
    for sh in (8, 4, 2, 1):
        v = v + v[idx ^ sh]
    return v


def _entmax_row(xbuf, obuf, cbuf, clist):
    lanes = lax.iota(jnp.int32, _L)
    pow2 = jnp.left_shift(jnp.ones((_L,), jnp.int32), lanes)

    def max_body(i, acc):
        for k in range(_U1):
            v = xbuf[pl.ds((i * _U1 + k) * _L, _L)] * 0.5
            xbuf[pl.ds((i * _U1 + k) * _L, _L)] = v
            acc = jnp.maximum(acc, v)
        return acc

    macc = lax.fori_loop(0, _NCHUNK // _U1, max_body, _splat(-jnp.inf))
    m = _bfly_max(macc)[0]
    thr = m - 1.0
    thrv = jnp.full((_L,), thr, dtype=jnp.float32)

    def filt_body(g, nl):
        vs = []
        cmv = _splat(-jnp.inf)
        for k in range(_GP):
            c0 = (g * _GP + k) * 2
            v0 = xbuf[pl.ds(c0 * _L, _L)]
            v1 = xbuf[pl.ds((c0 + 1) * _L, _L)]
            pm = _bfly_max(jnp.maximum(v0, v1))
            cmv = jnp.where(lanes == k, pm, cmv)
            vs.append((v0, v1))
        dirty = jnp.where(cmv > thrv, pow2, 0)
        bits = _bfly_sum(dirty)[0]
        for k in range(_GP):
            off = nl * (2 * _L)
            clist[pl.ds(nl, _L)] = jnp.full(
                (_L,), g * _GP + k, dtype=jnp.int32
            )
            cbuf[pl.ds(off, _L)] = vs[k][0]
            cbuf[pl.ds(off + _L, _L)] = vs[k][1]
            nl = nl + jnp.bitwise_and(jnp.right_shift(bits, k), 1)
        return nl

    nlist = lax.fori_loop(0, _NPAIR // _GP, filt_body, 0)
    clist[pl.ds(nlist, _L)] = jnp.full((_L,), _NPAIR, dtype=jnp.int32)
    for k in range(_U3):
        off = (nlist + k) * (2 * _L)
        cbuf[pl.ds(off, _L)] = _splat(-1e9)
        cbuf[pl.ds(off + _L, _L)] = _splat(-1e9)
    ntrip = (nlist + _U3 - 1) // _U3

    def newton_body(_, carry):
        lo, hi, tau = carry
        live = (hi - lo)[0] > _EPS
        ntrip_eff = jnp.where(live, ntrip, 0)

        def acc_body(i, ac):
            accf, accs = ac
            for k in range(_U3):
                off = (i * _U3 + k) * (2 * _L)
                y0 = jnp.maximum(cbuf[pl.ds(off, _L)] - tau, 0.0)
                y1 = jnp.maximum(cbuf[pl.ds(off + _L, _L)] - tau, 0.0)
                accf = accf + y0 * y0 + y1 * y1
                accs = accs + y0 + y1
            return accf, accs

        accf, accs = lax.fori_loop(
            0, ntrip_eff, acc_body, (_splat(0.0), _splat(0.0))
        )
        fv = _bfly_sum(accf)
        sv = _bfly_sum(accs)
        gt = fv > 1.0
        lo2 = jnp.where(gt, tau, lo)
        hi2 = jnp.where(gt, hi, tau)
        tn = tau + (fv - 1.0) / (2.0 * sv)
        mid = 0.5 * (lo2 + hi2)
        tn = jnp.where((tn > lo2) & (tn < hi2), tn, mid)
        g = jnp.full((_L,), jnp.where(live, 1.0, 0.0), dtype=jnp.float32)
        return (
            lo + g * (lo2 - lo),
            hi + g * (hi2 - hi),
            tau + g * (tn - tau),
        )

    mv = jnp.full((_L,), m, dtype=jnp.float32)
    _, _, tau = lax.fori_loop(
        0, _T, newton_body, (mv - 1.0, mv, mv - 0.5)
    )

    def out_body(g2, c):
        civ = clist[pl.ds(g2 * _L, _L)]
        for k in range(_L):
            j = g2 * _L + k
            ci = civ[k]
            off = j * (2 * _L)
            y0 = jnp.maximum(cbuf[pl.ds(off, _L)] - tau, 0.0)
            y1 = jnp.maximum(cbuf[pl.ds(off + _L, _L)] - tau, 0.0)
            obuf[pl.ds(ci * (2 * _L), _L)] = y0 * y0
            obuf[pl.ds(ci * (2 * _L) + _L, _L)] = y1 * y1
        return c

    p4trip = (nlist + _L - 1) // _L
    lax.fori_loop(0, p4trip, out_body, 0)
    return nlist


def _make_sc_kernel():
    mesh = plsc.VectorSubcoreMesh(core_axis_name="c", subcore_axis_name="s")

    @functools.partial(
        pl.kernel,
        mesh=mesh,
        out_type=jax.ShapeDtypeStruct((_R, _N), jnp.float32),
        scratch_types=[
            pltpu.VMEM((_N,), jnp.float32),
            pltpu.VMEM((_N + 2 * _L,), jnp.float32),
            pltpu.VMEM((_N + _L * (2 * _L),), jnp.float32),
            pltpu.VMEM((_NPAIR + _L,), jnp.int32),
        ],
    )
    def entmax_sc(scores, out, xbuf, obuf, cbuf, clist):
        wid = lax.axis_index("s") * 2 + lax.axis_index("c")

        def zero_body(i, c):
            for k in range(_U1):
                obuf[pl.ds((i * _U1 + k) * _L, _L)] = _splat(0.0)
            return c

        lax.fori_loop(0, (_N + 2 * _L) // (_U1 * _L), zero_body, 0)

        def row_body(r, c):
            row = wid * _RPW + r
            pltpu.sync_copy(scores.at[row], xbuf)
            nlist = _entmax_row(xbuf, obuf, cbuf, clist)
            pltpu.sync_copy(obuf.at[pl.ds(0, _N)], out.at[row])

            def rezero_body(g2, c2):
                civ = clist[pl.ds(g2 * _L, _L)]
                for k in range(_L):
                    ci = civ[k]
                    obuf[pl.ds(ci * (2 * _L), _L)] = _splat(0.0)
                    obuf[pl.ds(ci * (2 * _L) + _L, _L)] = _splat(0.0)
                return c2

            lax.fori_loop(0, (nlist + _L - 1) // _L, rezero_body, 0)
            return c

        lax.fori_loop(0, _RPW, row_body, 0)

    return entmax_sc


_entmax_sc = _make_sc_kernel()


def kernel(scores):
    return _entmax_sc(scores)


# speedup vs baseline: 3.0376x; 1.0815x over previous
import functools

import jax
import jax.numpy as jnp
from jax import lax
from jax.experimental import pallas as pl
from jax.experimental.pallas import tpu as pltpu
from jax.experimental.pallas import tpu_sc as plsc

_L = 16
_N = 32768
_NCHUNK = _N // _L
_NPAIR = _NCHUNK // 2
_R = 128
_NW = 32
_RPW = _R // _NW
_T = 30
_EPS = 6e-7
_U1 = 8
_U3 = 4
_GP = 8


def _splat(v):
    return jnp.full((_L,), v, dtype=jnp.float32)


def _bfly_max(v):
    idx = lax.iota(jnp.int32, _L)
    for sh in (8, 4, 2, 1):
        v = jnp.maximum(v, v[idx ^ sh])
    return v


def _bfly_sum(v):
    idx = lax.iota(jnp.int32, _L)
    for sh in (8, 4, 2, 1):
        v = v + v[idx ^ sh]
    return v


def _entmax_row(xbuf, obuf, cbuf, clist):
    lanes = lax.iota(jnp.int32, _L)
    pow2 = jnp.left_shift(jnp.ones((_L,), jnp.int32), lanes)

    def max_body(i, acc):
        for k in range(_U1):
            v = xbuf[pl.ds((i * _U1 + k) * _L, _L)] * 0.5
            xbuf[pl.ds((i * _U1 + k) * _L, _L)] = v
            acc = jnp.maximum(acc, v)
        return acc

    macc = lax.fori_loop(0, _NCHUNK // _U1, max_body, _splat(-jnp.inf))
    m = _bfly_max(macc)[0]
    thr = m - 1.0
    thrv = jnp.full((_L,), thr, dtype=jnp.float32)

    def filt_body(g, nl):
        vs = []
        cmv = _splat(-jnp.inf)
        for k in range(_GP):
            c0 = (g * _GP + k) * 2
            v0 = xbuf[pl.ds(c0 * _L, _L)]
            v1 = xbuf[pl.ds((c0 + 1) * _L, _L)]
            pm = _bfly_max(jnp.maximum(v0, v1))
            cmv = jnp.where(lanes == k, pm, cmv)
            vs.append((v0, v1))
        dirty = jnp.where(cmv > thrv, pow2, 0)
        bits = _bfly_sum(dirty)[0]
        for k in range(_GP):
            off = nl * (2 * _L)
            clist[pl.ds(nl, _L)] = jnp.full(
                (_L,), g * _GP + k, dtype=jnp.int32
            )
            cbuf[pl.ds(off, _L)] = vs[k][0]
            cbuf[pl.ds(off + _L, _L)] = vs[k][1]
            nl = nl + jnp.bitwise_and(jnp.right_shift(bits, k), 1)
        return nl

    nlist = lax.fori_loop(0, _NPAIR // _GP, filt_body, 0)
    clist[pl.ds(nlist, _L)] = jnp.full((_L,), _NPAIR, dtype=jnp.int32)
    for k in range(_U3):
        off = (nlist + k) * (2 * _L)
        cbuf[pl.ds(off, _L)] = _splat(-1e9)
        cbuf[pl.ds(off + _L, _L)] = _splat(-1e9)
    ntrip = (nlist + _U3 - 1) // _U3

    def newton_body(_, carry):
        lo, hi, tau = carry
        live = (hi - lo)[0] > _EPS
        ntrip_eff = jnp.where(live, ntrip, 0)

        def acc_body(i, ac):
            f0, f1, f2, f3, s0, s1 = ac
            off = i * (_U3 * 2 * _L)
            y0 = jnp.maximum(cbuf[pl.ds(off, _L)] - tau, 0.0)
            y1 = jnp.maximum(cbuf[pl.ds(off + _L, _L)] - tau, 0.0)
            y2 = jnp.maximum(cbuf[pl.ds(off + 2 * _L, _L)] - tau, 0.0)
            y3 = jnp.maximum(cbuf[pl.ds(off + 3 * _L, _L)] - tau, 0.0)
            y4 = jnp.maximum(cbuf[pl.ds(off + 4 * _L, _L)] - tau, 0.0)
            y5 = jnp.maximum(cbuf[pl.ds(off + 5 * _L, _L)] - tau, 0.0)
            y6 = jnp.maximum(cbuf[pl.ds(off + 6 * _L, _L)] - tau, 0.0)
            y7 = jnp.maximum(cbuf[pl.ds(off + 7 * _L, _L)] - tau, 0.0)
            f0 = f0 + y0 * y0 + y4 * y4
            f1 = f1 + y1 * y1 + y5 * y5
            f2 = f2 + y2 * y2 + y6 * y6
            f3 = f3 + y3 * y3 + y7 * y7
            s0 = s0 + y0 + y1 + y2 + y3
            s1 = s1 + y4 + y5 + y6 + y7
            return f0, f1, f2, f3, s0, s1

        z = _splat(0.0)
        f0, f1, f2, f3, s0, s1 = lax.fori_loop(
            0, ntrip_eff, acc_body, (z, z, z, z, z, z)
        )
        fv = _bfly_sum((f0 + f1) + (f2 + f3))
        sv = _bfly_sum(s0 + s1)
        gt = fv > 1.0
        lo2 = jnp.where(gt, tau, lo)
        hi2 = jnp.where(gt, hi, tau)
        tn = tau + (fv - 1.0) / (2.0 * sv)
        mid = 0.5 * (lo2 + hi2)
        tn = jnp.where((tn > lo2) & (tn < hi2), tn, mid)
        g = jnp.full((_L,), jnp.where(live, 1.0, 0.0), dtype=jnp.float32)
        return (
            lo + g * (lo2 - lo),
            hi + g * (hi2 - hi),
            tau + g * (tn - tau),
        )

    mv = jnp.full((_L,), m, dtype=jnp.float32)
    _, _, tau = lax.fori_loop(
        0, _T, newton_body, (mv - 1.0, mv, mv - 0.5)
    )

    def out_body(g2, c):
        civ = clist[pl.ds(g2 * _L, _L)]
        for k in range(_L):
            j = g2 * _L + k
            ci = civ[k]
            off = j * (2 * _L)
            y0 = jnp.maximum(cbuf[pl.ds(off, _L)] - tau, 0.0)
            y1 = jnp.maximum(cbuf[pl.ds(off + _L, _L)] - tau, 0.0)
            obuf[pl.ds(ci * (2 * _L), _L)] = y0 * y0
            obuf[pl.ds(ci * (2 * _L) + _L, _L)] = y1 * y1
        return c

    p4trip = (nlist + _L - 1) // _L
    lax.fori_loop(0, p4trip, out_body, 0)
    return nlist


def _make_sc_kernel():
    mesh = plsc.VectorSubcoreMesh(core_axis_name="c", subcore_axis_name="s")

    @functools.partial(
        pl.kernel,
        mesh=mesh,
        out_type=jax.ShapeDtypeStruct((_R, _N), jnp.float32),
        scratch_types=[
            pltpu.VMEM((_N,), jnp.float32),
            pltpu.VMEM((_N + 2 * _L,), jnp.float32),
            pltpu.VMEM((_N + _L * (2 * _L),), jnp.float32),
            pltpu.VMEM((_NPAIR + _L,), jnp.int32),
        ],
    )
    def entmax_sc(scores, out, xbuf, obuf, cbuf, clist):
        wid = lax.axis_index("s") * 2 + lax.axis_index("c")

        def zero_body(i, c):
            for k in range(_U1):
                obuf[pl.ds((i * _U1 + k) * _L, _L)] = _splat(0.0)
            return c

        lax.fori_loop(0, (_N + 2 * _L) // (_U1 * _L), zero_body, 0)

        def row_body(r, c):
            row = wid * _RPW + r
            pltpu.sync_copy(scores.at[row], xbuf)
            nlist = _entmax_row(xbuf, obuf, cbuf, clist)
            pltpu.sync_copy(obuf.at[pl.ds(0, _N)], out.at[row])

            def rezero_body(g2, c2):
                civ = clist[pl.ds(g2 * _L, _L)]
                for k in range(_L):
                    ci = civ[k]
                    obuf[pl.ds(ci * (2 * _L), _L)] = _splat(0.0)
                    obuf[pl.ds(ci * (2 * _L) + _L, _L)] = _splat(0.0)
                return c2

            lax.fori_loop(0, (nlist + _L - 1) // _L, rezero_body, 0)
            return c

        lax.fori_loop(0, _RPW, row_body, 0)

    return entmax_sc


_entmax_sc = _make_sc_kernel()


def kernel(scores):
    return _entmax_sc(scores)


# speedup vs baseline: 5.1170x; 1.6845x over previous
import functools

import jax
import jax.numpy as jnp
from jax import lax
from jax.experimental import pallas as pl
from jax.experimental.pallas import tpu as pltpu
from jax.experimental.pallas import tpu_sc as plsc

_L = 16
_N = 32768
_NCHUNK = _N // _L
_NPAIR = _NCHUNK // 2
_R = 128
_NW = 32
_RPW = _R // _NW
_T = 30
_EPS = 6e-7
_FEPS = 3e-5
_U1 = 8
_U3 = 4
_GP = 8


def _splat(v):
    return jnp.full((_L,), v, dtype=jnp.float32)


def _bfly_max(v):
    idx = lax.iota(jnp.int32, _L)
    for sh in (8, 4, 2, 1):
        v = jnp.maximum(v, v[idx ^ sh])
    return v


def _bfly_sum(v):
    idx = lax.iota(jnp.int32, _L)
    for sh in (8, 4, 2, 1):
        v = v + v[idx ^ sh]
    return v


def _entmax_row(xbuf, obuf, cbuf, clist):
    lanes = lax.iota(jnp.int32, _L)
    pow2 = jnp.left_shift(jnp.ones((_L,), jnp.int32), lanes)

    def max_body(i, acc):
        for k in range(_U1):
            v = xbuf[pl.ds((i * _U1 + k) * _L, _L)] * 0.5
            xbuf[pl.ds((i * _U1 + k) * _L, _L)] = v
            acc = jnp.maximum(acc, v)
        return acc

    macc = lax.fori_loop(0, _NCHUNK // _U1, max_body, _splat(-jnp.inf))
    m = _bfly_max(macc)[0]
    thr = m - 1.0
    thrv = jnp.full((_L,), thr, dtype=jnp.float32)

    def filt_body(g, nl):
        vs = []
        cmv = _splat(-jnp.inf)
        for k in range(_GP):
            c0 = (g * _GP + k) * 2
            v0 = xbuf[pl.ds(c0 * _L, _L)]
            v1 = xbuf[pl.ds((c0 + 1) * _L, _L)]
            pm = _bfly_max(jnp.maximum(v0, v1))
            cmv = jnp.where(lanes == k, pm, cmv)
            vs.append((v0, v1))
        dirty = jnp.where(cmv > thrv, pow2, 0)
        bits = _bfly_sum(dirty)[0]
        for k in range(_GP):
            off = nl * (2 * _L)
            clist[pl.ds(nl, _L)] = jnp.full(
                (_L,), g * _GP + k, dtype=jnp.int32
            )
            cbuf[pl.ds(off, _L)] = vs[k][0]
            cbuf[pl.ds(off + _L, _L)] = vs[k][1]
            nl = nl + jnp.bitwise_and(jnp.right_shift(bits, k), 1)
        return nl

    nlist = lax.fori_loop(0, _NPAIR // _GP, filt_body, 0)
    clist[pl.ds(nlist, _L)] = jnp.full((_L,), _NPAIR, dtype=jnp.int32)
    for k in range(_U3):
        off = (nlist + k) * (2 * _L)
        cbuf[pl.ds(off, _L)] = _splat(-1e9)
        cbuf[pl.ds(off + _L, _L)] = _splat(-1e9)
    ntrip = (nlist + _U3 - 1) // _U3

    def newton_body(_, carry):
        lo, hi, tau, fdev = carry
        live = jnp.logical_and((hi - lo)[0] > _EPS, fdev[0] > _FEPS)
        ntrip_eff = jnp.where(live, ntrip, 0)

        def acc_body(i, ac):
            f0, f1, f2, f3, s0, s1 = ac
            off = i * (_U3 * 2 * _L)
            y0 = jnp.maximum(cbuf[pl.ds(off, _L)] - tau, 0.0)
            y1 = jnp.maximum(cbuf[pl.ds(off + _L, _L)] - tau, 0.0)
            y2 = jnp.maximum(cbuf[pl.ds(off + 2 * _L, _L)] - tau, 0.0)
            y3 = jnp.maximum(cbuf[pl.ds(off + 3 * _L, _L)] - tau, 0.0)
            y4 = jnp.maximum(cbuf[pl.ds(off + 4 * _L, _L)] - tau, 0.0)
            y5 = jnp.maximum(cbuf[pl.ds(off + 5 * _L, _L)] - tau, 0.0)
            y6 = jnp.maximum(cbuf[pl.ds(off + 6 * _L, _L)] - tau, 0.0)
            y7 = jnp.maximum(cbuf[pl.ds(off + 7 * _L, _L)] - tau, 0.0)
            f0 = f0 + y0 * y0 + y4 * y4
            f1 = f1 + y1 * y1 + y5 * y5
            f2 = f2 + y2 * y2 + y6 * y6
            f3 = f3 + y3 * y3 + y7 * y7
            s0 = s0 + y0 + y1 + y2 + y3
            s1 = s1 + y4 + y5 + y6 + y7
            return f0, f1, f2, f3, s0, s1

        z = _splat(0.0)
        f0, f1, f2, f3, s0, s1 = lax.fori_loop(
            0, ntrip_eff, acc_body, (z, z, z, z, z, z)
        )
        fv = _bfly_sum((f0 + f1) + (f2 + f3))
        sv = _bfly_sum(s0 + s1)
        gt = fv > 1.0
        lo2 = jnp.where(gt, tau, lo)
        hi2 = jnp.where(gt, hi, tau)
        tn = tau + (fv - 1.0) / (2.0 * sv)
        mid = 0.5 * (lo2 + hi2)
        tn = jnp.where((tn > lo2) & (tn < hi2), tn, mid)
        g = jnp.full((_L,), jnp.where(live, 1.0, 0.0), dtype=jnp.float32)
        fd = jnp.abs(fv - 1.0)
        return (
            lo + g * (lo2 - lo),
            hi + g * (hi2 - hi),
            tau + g * (tn - tau),
            fdev + g * (fd - fdev),
        )

    mv = jnp.full((_L,), m, dtype=jnp.float32)
    _, _, tau, _ = lax.fori_loop(
        0, _T, newton_body, (mv - 1.0, mv, mv - 0.5, _splat(1.0))
    )

    def out_body(g2, c):
        civ = clist[pl.ds(g2 * _L, _L)]
        for k in range(_L):
            j = g2 * _L + k
            ci = civ[k]
            off = j * (2 * _L)
            y0 = jnp.maximum(cbuf[pl.ds(off, _L)] - tau, 0.0)
            y1 = jnp.maximum(cbuf[pl.ds(off + _L, _L)] - tau, 0.0)
            obuf[pl.ds(ci * (2 * _L), _L)] = y0 * y0
            obuf[pl.ds(ci * (2 * _L) + _L, _L)] = y1 * y1
        return c

    p4trip = (nlist + _L - 1) // _L
    lax.fori_loop(0, p4trip, out_body, 0)
    return nlist


def _make_sc_kernel():
    mesh = plsc.VectorSubcoreMesh(core_axis_name="c", subcore_axis_name="s")

    @functools.partial(
        pl.kernel,
        mesh=mesh,
        out_type=jax.ShapeDtypeStruct((_R, _N), jnp.float32),
        scratch_types=[
            pltpu.VMEM((_N,), jnp.float32),
            pltpu.VMEM((_N + 2 * _L,), jnp.float32),
            pltpu.VMEM((_N + _L * (2 * _L),), jnp.float32),
            pltpu.VMEM((_NPAIR + _L,), jnp.int32),
        ],
    )
    def entmax_sc(scores, out, xbuf, obuf, cbuf, clist):
        wid = lax.axis_index("s") * 2 + lax.axis_index("c")

        def zero_body(i, c):
            for k in range(_U1):
                obuf[pl.ds((i * _U1 + k) * _L, _L)] = _splat(0.0)
            return c

        lax.fori_loop(0, (_N + 2 * _L) // (_U1 * _L), zero_body, 0)

        def row_body(r, c):
            row = wid * _RPW + r
            pltpu.sync_copy(scores.at[row], xbuf)
            nlist = _entmax_row(xbuf, obuf, cbuf, clist)
            pltpu.sync_copy(obuf.at[pl.ds(0, _N)], out.at[row])

            def rezero_body(g2, c2):
                civ = clist[pl.ds(g2 * _L, _L)]
                for k in range(_L):
                    ci = civ[k]
                    obuf[pl.ds(ci * (2 * _L), _L)] = _splat(0.0)
                    obuf[pl.ds(ci * (2 * _L) + _L, _L)] = _splat(0.0)
                return c2

            lax.fori_loop(0, (nlist + _L - 1) // _L, rezero_body, 0)
            return c

        lax.fori_loop(0, _RPW, row_body, 0)

    return entmax_sc


_entmax_sc = _make_sc_kernel()


def kernel(scores):
    return _entmax_sc(scores)
